# baseline (speedup 1.0000x reference)
import functools

import jax
import jax.numpy as jnp
from jax import lax
from jax.experimental import pallas as pl
from jax.experimental.pallas import tpu as pltpu
from jax.experimental.pallas import tpu_sc as plsc

N = 1_000_000
C_SIZE = 64
KEY_SIZE = 32
B = 64
TOPK = 64
VALUE_SIZE = 1 + 1 + C_SIZE
QKV_SIZE = 2 * KEY_SIZE + VALUE_SIZE

BN = 8000
NB = N // BN


def _topk_body(c_ref, mem_ref, vals_ref, idx_ref):
    t = pl.program_id(0)

    @pl.when(t == 0)
    def _init():
        vals_ref[...] = jnp.full((B, TOPK), -jnp.inf, jnp.float32)
        idx_ref[...] = jnp.zeros((B, TOPK), jnp.int32)

    s = lax.dot_general(
        c_ref[...], mem_ref[...], (((1,), (1,)), ((), ())),
        preferred_element_type=jnp.float32,
    )
    col = lax.broadcasted_iota(jnp.int32, (B, BN), 1)
    kiota = lax.broadcasted_iota(jnp.int32, (B, TOPK), 1)
    m0 = jnp.max(s, axis=1, keepdims=True)

    def cond(carry):
        _, m_, rv_, _ = carry
        return jnp.any(m_ > jnp.min(rv_, axis=1, keepdims=True))

    def body(carry):
        s_, m_, rv_, ri_ = carry
        a = jnp.argmax(s_, axis=1).reshape(B, 1)
        rmin = jnp.min(rv_, axis=1, keepdims=True)
        p = jnp.argmin(rv_, axis=1).reshape(B, 1)
        sel = (kiota == p) & (m_ > rmin)
        rv_ = jnp.where(sel, m_, rv_)
        ri_ = jnp.where(sel, t * BN + a, ri_)
        s_ = jnp.where(col == a, -jnp.inf, s_)
        m_ = jnp.max(s_, axis=1, keepdims=True)
        return s_, m_, rv_, ri_

    _, _, rv, ri = lax.while_loop(
        cond, body, (s, m0, vals_ref[...], idx_ref[...]))
    vals_ref[...] = rv
    idx_ref[...] = ri


def _run_topk(c, mem_c):
    return pl.pallas_call(
        _topk_body,
        grid=(NB,),
        in_specs=[
            pl.BlockSpec((B, C_SIZE), lambda t: (0, 0)),
            pl.BlockSpec((BN, C_SIZE), lambda t: (t, 0)),
        ],
        out_specs=[
            pl.BlockSpec((B, TOPK), lambda t: (0, 0)),
            pl.BlockSpec((B, TOPK), lambda t: (0, 0)),
        ],
        out_shape=[
            jax.ShapeDtypeStruct((B, TOPK), jnp.float32),
            jax.ShapeDtypeStruct((B, TOPK), jnp.int32),
        ],
    )(c, mem_c)


_NC = 2
_NS = 16
_NW = _NC * _NS
_ROWS = B * TOPK
_RPW = _ROWS // _NW


_TW = 16
_L = 16


def _gather_body(mem_x_hbm, mem_t16_hbm, idx_hbm, x_out, t_out,
                 idx_v, rows_v, xrows_v, trows_v, tsel_v, sem):
    wid = lax.axis_index("s") * _NC + lax.axis_index("c")
    base = wid * _RPW
    pltpu.sync_copy(idx_hbm.at[pl.ds(base, _RPW)], idx_v)
    for j in range(_RPW // _L):
        rows_v[pl.ds(j * _L, _L)] = lax.shift_right_logical(
            idx_v[pl.ds(j * _L, _L)], 4)
    pltpu.async_copy(mem_x_hbm.at[idx_v], xrows_v, sem).wait()
    pltpu.async_copy(mem_t16_hbm.at[rows_v], trows_v, sem).wait()
    for j in range(_RPW // _L):
        lanes = jnp.bitwise_and(idx_v[pl.ds(j * _L, _L)], _TW - 1)
        rids = jax.lax.iota(jnp.int32, _L) + j * _L
        tsel_v[pl.ds(j * _L, _L)] = plsc.load_gather(trows_v, [rids, lanes])
    pltpu.sync_copy(xrows_v, x_out.at[pl.ds(base, _RPW)])
    pltpu.sync_copy(tsel_v, t_out.at[pl.ds(base, _RPW)])


def _run_gather(idx_flat, mem_x, mem_t16):
    mesh = plsc.VectorSubcoreMesh(core_axis_name="c", subcore_axis_name="s")
    f = functools.partial(
        pl.kernel,
        mesh=mesh,
        out_type=[
            jax.ShapeDtypeStruct((_ROWS, C_SIZE), jnp.float32),
            jax.ShapeDtypeStruct((_ROWS,), jnp.float32),
        ],
        scratch_types=[
            pltpu.VMEM((_RPW,), jnp.int32),
            pltpu.VMEM((_RPW,), jnp.int32),
            pltpu.VMEM((_RPW, C_SIZE), jnp.float32),
            pltpu.VMEM((_RPW, _TW), jnp.float32),
            pltpu.VMEM((_RPW,), jnp.float32),
            pltpu.SemaphoreType.DMA,
        ],
        compiler_params=pltpu.CompilerParams(
            use_tc_tiling_on_sc=False, needs_layout_passes=False),
    )(_gather_body)
    return f(mem_x, mem_t16, idx_flat)


def _ln(x, g, b, eps=1e-5):
    mu = jnp.mean(x, axis=-1, keepdims=True)
    xc = x - mu
    var = jnp.mean(xc * xc, axis=-1, keepdims=True)
    return xc / jnp.sqrt(var + eps) * g + b


def _head_body(vals_ref, t_ref, x_ref, wqkv_ref, bqkv_ref, gq_ref, bq_ref,
               gm_ref, bm_ref, w1_ref, b1_ref, w2_ref, b2_ref, p1_ref,
               pb1_ref, p2_ref, pb2_ref, out_ref, qkv_sr, att_sr):
    md = jnp.concatenate([vals_ref[...], t_ref[...], x_ref[...]], axis=1)

    qkv = jnp.dot(md, wqkv_ref[...],
                  preferred_element_type=jnp.float32) + bqkv_ref[...]
    qkv = _ln(qkv, gq_ref[...], bq_ref[...])
    qkv_sr[...] = qkv

    scale = KEY_SIZE ** -0.5

    def attend(b, _):
        blk = qkv_sr[pl.ds(b * TOPK, TOPK), :]
        q = blk[:, :KEY_SIZE] * scale
        kk = blk[:, KEY_SIZE:2 * KEY_SIZE]
        v = blk[:, 2 * KEY_SIZE:]
        w = jnp.dot(q, kk.T, preferred_element_type=jnp.float32)
        w = w - jnp.max(w, axis=1, keepdims=True)
        w = jnp.exp(w)
        w = w / jnp.sum(w, axis=1, keepdims=True)
        att_sr[pl.ds(b * TOPK, TOPK), :] = jnp.dot(
            w, v, preferred_element_type=jnp.float32)
        return 0

    lax.fori_loop(0, B, attend, 0)

    gm = gm_ref[...]
    bm = bm_ref[...]
    mem = _ln(md + att_sr[...], gm, bm)
    h = jnp.dot(jnp.maximum(jnp.dot(mem, w1_ref[...],
                                    preferred_element_type=jnp.float32)
                            + b1_ref[...], 0.0),
                w2_ref[...], preferred_element_type=jnp.float32) + b2_ref[...]
    mem = _ln(h + mem, gm, bm)

    rows = lax.broadcasted_iota(jnp.int32, (B, _ROWS), 1)
    qsel = lax.broadcasted_iota(jnp.int32, (B, _ROWS), 0)
    pool = jnp.where(rows // TOPK == qsel, 1.0 / TOPK, 0.0)
    pooled = jnp.dot(pool, mem, preferred_element_type=jnp.float32)

    c_prime = jnp.dot(
        jnp.maximum(jnp.dot(pooled, p1_ref[...],
                            preferred_element_type=jnp.float32)
                    + pb1_ref[...], 0.0),
        p2_ref[...], preferred_element_type=jnp.float32) + pb2_ref[...]
    out_ref[...] = c_prime


def _run_head(vals, tvals, x_rows, W_qkv, b_qkv, ln_qkv_g, ln_qkv_b,
              ln_mem_g, ln_mem_b, W_mlp1, b_mlp1, W_mlp2, b_mlp2,
              W_proj1, b_proj1, W_proj2, b_proj2):
    args = (vals, tvals, x_rows, W_qkv, b_qkv.reshape(1, -1),
            ln_qkv_g.reshape(1, -1), ln_qkv_b.reshape(1, -1),
            ln_mem_g.reshape(1, -1), ln_mem_b.reshape(1, -1),
            W_mlp1, b_mlp1.reshape(1, -1), W_mlp2, b_mlp2.reshape(1, -1),
            W_proj1, b_proj1.reshape(1, -1), W_proj2, b_proj2.reshape(1, -1))
    return pl.pallas_call(
        _head_body,
        out_shape=jax.ShapeDtypeStruct((B, C_SIZE), jnp.float32),
        scratch_shapes=[
            pltpu.VMEM((_ROWS, QKV_SIZE), jnp.float32),
            pltpu.VMEM((_ROWS, VALUE_SIZE), jnp.float32),
        ],
    )(*args)


def kernel(c, k, mem_c, mem_t, mem_x, W_qkv, b_qkv, ln_qkv_g, ln_qkv_b,
           ln_mem_g, ln_mem_b, W_mlp1, b_mlp1, W_mlp2, b_mlp2,
           W_proj1, b_proj1, W_proj2, b_proj2):
    vals, idx = _run_topk(c, mem_c)
    x_rows, t_flat = _run_gather(idx.reshape(_ROWS), mem_x,
                                 mem_t.reshape(N // _TW, _TW))
    vals = vals.reshape(_ROWS, 1)
    tvals = t_flat.reshape(_ROWS, 1)
    return _run_head(vals, tvals, x_rows, W_qkv, b_qkv, ln_qkv_g, ln_qkv_b,
                     ln_mem_g, ln_mem_b, W_mlp1, b_mlp1, W_mlp2, b_mlp2,
                     W_proj1, b_proj1, W_proj2, b_proj2)


# speedup vs baseline: 1.0639x; 1.0639x over previous
import functools

import jax
import jax.numpy as jnp
from jax import lax
from jax.experimental import pallas as pl
from jax.experimental.pallas import tpu as pltpu
from jax.experimental.pallas import tpu_sc as plsc

N = 1_000_000
C_SIZE = 64
KEY_SIZE = 32
B = 64
TOPK = 64
VALUE_SIZE = 1 + 1 + C_SIZE
QKV_SIZE = 2 * KEY_SIZE + VALUE_SIZE

BN = 5000
NB = N // BN


def _topk_body(c_ref, mem_ref, vals_ref, idx_ref):
    t = pl.program_id(0)

    @pl.when(t == 0)
    def _init():
        vals_ref[...] = jnp.full((B, TOPK), -jnp.inf, jnp.float32)
        idx_ref[...] = jnp.zeros((B, TOPK), jnp.int32)

    s = lax.dot_general(
        c_ref[...], mem_ref[...], (((1,), (1,)), ((), ())),
        preferred_element_type=jnp.float32,
    )
    col = lax.broadcasted_iota(jnp.int32, (B, BN), 1)
    kiota = lax.broadcasted_iota(jnp.int32, (B, TOPK), 1)
    m0 = jnp.max(s, axis=1, keepdims=True)

    def cond(carry):
        _, m_, rv_, _ = carry
        return jnp.any(m_ > jnp.min(rv_, axis=1, keepdims=True))

    def body(carry):
        s_, m_, rv_, ri_ = carry
        a = jnp.argmax(s_, axis=1).reshape(B, 1)
        rmin = jnp.min(rv_, axis=1, keepdims=True)
        p = jnp.argmin(rv_, axis=1).reshape(B, 1)
        sel = (kiota == p) & (m_ > rmin)
        rv_ = jnp.where(sel, m_, rv_)
        ri_ = jnp.where(sel, t * BN + a, ri_)
        s_ = jnp.where(col == a, -jnp.inf, s_)
        m_ = jnp.max(s_, axis=1, keepdims=True)
        return s_, m_, rv_, ri_

    _, _, rv, ri = lax.while_loop(
        cond, body, (s, m0, vals_ref[...], idx_ref[...]))
    vals_ref[...] = rv
    idx_ref[...] = ri


def _run_topk(c, mem_c):
    return pl.pallas_call(
        _topk_body,
        grid=(NB,),
        in_specs=[
            pl.BlockSpec((B, C_SIZE), lambda t: (0, 0)),
            pl.BlockSpec((BN, C_SIZE), lambda t: (t, 0)),
        ],
        out_specs=[
            pl.BlockSpec((B, TOPK), lambda t: (0, 0)),
            pl.BlockSpec((B, TOPK), lambda t: (0, 0)),
        ],
        out_shape=[
            jax.ShapeDtypeStruct((B, TOPK), jnp.float32),
            jax.ShapeDtypeStruct((B, TOPK), jnp.int32),
        ],
    )(c, mem_c)


_NC = 2
_NS = 16
_NW = _NC * _NS
_ROWS = B * TOPK
_RPW = _ROWS // _NW


_TW = 16
_L = 16


def _gather_body(mem_x_hbm, mem_t16_hbm, idx_hbm, x_out, t_out,
                 idx_v, rows_v, xrows_v, trows_v, tsel_v, sem):
    wid = lax.axis_index("s") * _NC + lax.axis_index("c")
    base = wid * _RPW
    pltpu.sync_copy(idx_hbm.at[pl.ds(base, _RPW)], idx_v)
    for j in range(_RPW // _L):
        rows_v[pl.ds(j * _L, _L)] = lax.shift_right_logical(
            idx_v[pl.ds(j * _L, _L)], 4)
    pltpu.async_copy(mem_x_hbm.at[idx_v], xrows_v, sem).wait()
    pltpu.async_copy(mem_t16_hbm.at[rows_v], trows_v, sem).wait()
    for j in range(_RPW // _L):
        lanes = jnp.bitwise_and(idx_v[pl.ds(j * _L, _L)], _TW - 1)
        rids = jax.lax.iota(jnp.int32, _L) + j * _L
        tsel_v[pl.ds(j * _L, _L)] = plsc.load_gather(trows_v, [rids, lanes])
    pltpu.sync_copy(xrows_v, x_out.at[pl.ds(base, _RPW)])
    pltpu.sync_copy(tsel_v, t_out.at[pl.ds(base, _RPW)])


def _run_gather(idx_flat, mem_x, mem_t16):
    mesh = plsc.VectorSubcoreMesh(core_axis_name="c", subcore_axis_name="s")
    f = functools.partial(
        pl.kernel,
        mesh=mesh,
        out_type=[
            jax.ShapeDtypeStruct((_ROWS, C_SIZE), jnp.float32),
            jax.ShapeDtypeStruct((_ROWS,), jnp.float32),
        ],
        scratch_types=[
            pltpu.VMEM((_RPW,), jnp.int32),
            pltpu.VMEM((_RPW,), jnp.int32),
            pltpu.VMEM((_RPW, C_SIZE), jnp.float32),
            pltpu.VMEM((_RPW, _TW), jnp.float32),
            pltpu.VMEM((_RPW,), jnp.float32),
            pltpu.SemaphoreType.DMA,
        ],
        compiler_params=pltpu.CompilerParams(
            use_tc_tiling_on_sc=False, needs_layout_passes=False),
    )(_gather_body)
    return f(mem_x, mem_t16, idx_flat)


def _ln(x, g, b, eps=1e-5):
    mu = jnp.mean(x, axis=-1, keepdims=True)
    xc = x - mu
    var = jnp.mean(xc * xc, axis=-1, keepdims=True)
    return xc / jnp.sqrt(var + eps) * g + b


def _head_body(vals_ref, t_ref, x_ref, wqkv_ref, bqkv_ref, gq_ref, bq_ref,
               gm_ref, bm_ref, w1_ref, b1_ref, w2_ref, b2_ref, p1_ref,
               pb1_ref, p2_ref, pb2_ref, out_ref, qkv_sr, att_sr):
    md = jnp.concatenate([vals_ref[...], t_ref[...], x_ref[...]], axis=1)

    qkv = jnp.dot(md, wqkv_ref[...],
                  preferred_element_type=jnp.float32) + bqkv_ref[...]
    qkv = _ln(qkv, gq_ref[...], bq_ref[...])
    qkv_sr[...] = qkv

    scale = KEY_SIZE ** -0.5

    def attend(b, _):
        blk = qkv_sr[pl.ds(b * TOPK, TOPK), :]
        q = blk[:, :KEY_SIZE] * scale
        kk = blk[:, KEY_SIZE:2 * KEY_SIZE]
        v = blk[:, 2 * KEY_SIZE:]
        w = jnp.dot(q, kk.T, preferred_element_type=jnp.float32)
        w = w - jnp.max(w, axis=1, keepdims=True)
        w = jnp.exp(w)
        w = w / jnp.sum(w, axis=1, keepdims=True)
        att_sr[pl.ds(b * TOPK, TOPK), :] = jnp.dot(
            w, v, preferred_element_type=jnp.float32)
        return 0

    lax.fori_loop(0, B, attend, 0)

    gm = gm_ref[...]
    bm = bm_ref[...]
    mem = _ln(md + att_sr[...], gm, bm)
    h = jnp.dot(jnp.maximum(jnp.dot(mem, w1_ref[...],
                                    preferred_element_type=jnp.float32)
                            + b1_ref[...], 0.0),
                w2_ref[...], preferred_element_type=jnp.float32) + b2_ref[...]
    mem = _ln(h + mem, gm, bm)

    rows = lax.broadcasted_iota(jnp.int32, (B, _ROWS), 1)
    qsel = lax.broadcasted_iota(jnp.int32, (B, _ROWS), 0)
    pool = jnp.where(rows // TOPK == qsel, 1.0 / TOPK, 0.0)
    pooled = jnp.dot(pool, mem, preferred_element_type=jnp.float32)

    c_prime = jnp.dot(
        jnp.maximum(jnp.dot(pooled, p1_ref[...],
                            preferred_element_type=jnp.float32)
                    + pb1_ref[...], 0.0),
        p2_ref[...], preferred_element_type=jnp.float32) + pb2_ref[...]
    out_ref[...] = c_prime


def _run_head(vals, tvals, x_rows, W_qkv, b_qkv, ln_qkv_g, ln_qkv_b,
              ln_mem_g, ln_mem_b, W_mlp1, b_mlp1, W_mlp2, b_mlp2,
              W_proj1, b_proj1, W_proj2, b_proj2):
    args = (vals, tvals, x_rows, W_qkv, b_qkv.reshape(1, -1),
            ln_qkv_g.reshape(1, -1), ln_qkv_b.reshape(1, -1),
            ln_mem_g.reshape(1, -1), ln_mem_b.reshape(1, -1),
            W_mlp1, b_mlp1.reshape(1, -1), W_mlp2, b_mlp2.reshape(1, -1),
            W_proj1, b_proj1.reshape(1, -1), W_proj2, b_proj2.reshape(1, -1))
    return pl.pallas_call(
        _head_body,
        out_shape=jax.ShapeDtypeStruct((B, C_SIZE), jnp.float32),
        scratch_shapes=[
            pltpu.VMEM((_ROWS, QKV_SIZE), jnp.float32),
            pltpu.VMEM((_ROWS, VALUE_SIZE), jnp.float32),
        ],
    )(*args)


def kernel(c, k, mem_c, mem_t, mem_x, W_qkv, b_qkv, ln_qkv_g, ln_qkv_b,
           ln_mem_g, ln_mem_b, W_mlp1, b_mlp1, W_mlp2, b_mlp2,
           W_proj1, b_proj1, W_proj2, b_proj2):
    vals, idx = _run_topk(c, mem_c)
    x_rows, t_flat = _run_gather(idx.reshape(_ROWS), mem_x,
                                 mem_t.reshape(N // _TW, _TW))
    vals = vals.reshape(_ROWS, 1)
    tvals = t_flat.reshape(_ROWS, 1)
    return _run_head(vals, tvals, x_rows, W_qkv, b_qkv, ln_qkv_g, ln_qkv_b,
                     ln_mem_g, ln_mem_b, W_mlp1, b_mlp1, W_mlp2, b_mlp2,
                     W_proj1, b_proj1, W_proj2, b_proj2)


# speedup vs baseline: 1.0950x; 1.0293x over previous
import functools

import jax
import jax.numpy as jnp
from jax import lax
from jax.experimental import pallas as pl
from jax.experimental.pallas import tpu as pltpu
from jax.experimental.pallas import tpu_sc as plsc

N = 1_000_000
C_SIZE = 64
KEY_SIZE = 32
B = 64
TOPK = 64
VALUE_SIZE = 1 + 1 + C_SIZE
QKV_SIZE = 2 * KEY_SIZE + VALUE_SIZE

BN = 4000
NB = N // BN


def _topk_body(c_ref, mem_ref, vals_ref, idx_ref):
    t = pl.program_id(0)

    @pl.when(t == 0)
    def _init():
        vals_ref[...] = jnp.full((B, TOPK), -jnp.inf, jnp.float32)
        idx_ref[...] = jnp.zeros((B, TOPK), jnp.int32)

    s = lax.dot_general(
        c_ref[...], mem_ref[...], (((1,), (1,)), ((), ())),
        preferred_element_type=jnp.float32,
    )
    col = lax.broadcasted_iota(jnp.int32, (B, BN), 1)
    kiota = lax.broadcasted_iota(jnp.int32, (B, TOPK), 1)
    m0 = jnp.max(s, axis=1, keepdims=True)

    def cond(carry):
        _, m_, rv_, _ = carry
        return jnp.any(m_ > jnp.min(rv_, axis=1, keepdims=True))

    def body(carry):
        s_, m_, rv_, ri_ = carry
        a = jnp.argmax(s_, axis=1).reshape(B, 1)
        rmin = jnp.min(rv_, axis=1, keepdims=True)
        p = jnp.argmin(rv_, axis=1).reshape(B, 1)
        sel = (kiota == p) & (m_ > rmin)
        rv_ = jnp.where(sel, m_, rv_)
        ri_ = jnp.where(sel, t * BN + a, ri_)
        s_ = jnp.where(col == a, -jnp.inf, s_)
        m_ = jnp.max(s_, axis=1, keepdims=True)
        return s_, m_, rv_, ri_

    _, _, rv, ri = lax.while_loop(
        cond, body, (s, m0, vals_ref[...], idx_ref[...]))
    vals_ref[...] = rv
    idx_ref[...] = ri


def _run_topk(c, mem_c):
    return pl.pallas_call(
        _topk_body,
        grid=(NB,),
        in_specs=[
            pl.BlockSpec((B, C_SIZE), lambda t: (0, 0)),
            pl.BlockSpec((BN, C_SIZE), lambda t: (t, 0)),
        ],
        out_specs=[
            pl.BlockSpec((B, TOPK), lambda t: (0, 0)),
            pl.BlockSpec((B, TOPK), lambda t: (0, 0)),
        ],
        out_shape=[
            jax.ShapeDtypeStruct((B, TOPK), jnp.float32),
            jax.ShapeDtypeStruct((B, TOPK), jnp.int32),
        ],
    )(c, mem_c)


_NC = 2
_NS = 16
_NW = _NC * _NS
_ROWS = B * TOPK
_RPW = _ROWS // _NW


_TW = 16
_L = 16


def _gather_body(mem_x_hbm, mem_t16_hbm, idx_hbm, x_out, t_out,
                 idx_v, rows_v, xrows_v, trows_v, tsel_v, sem):
    wid = lax.axis_index("s") * _NC + lax.axis_index("c")
    base = wid * _RPW
    pltpu.sync_copy(idx_hbm.at[pl.ds(base, _RPW)], idx_v)
    for j in range(_RPW // _L):
        rows_v[pl.ds(j * _L, _L)] = lax.shift_right_logical(
            idx_v[pl.ds(j * _L, _L)], 4)
    pltpu.async_copy(mem_x_hbm.at[idx_v], xrows_v, sem).wait()
    pltpu.async_copy(mem_t16_hbm.at[rows_v], trows_v, sem).wait()
    for j in range(_RPW // _L):
        lanes = jnp.bitwise_and(idx_v[pl.ds(j * _L, _L)], _TW - 1)
        rids = jax.lax.iota(jnp.int32, _L) + j * _L
        tsel_v[pl.ds(j * _L, _L)] = plsc.load_gather(trows_v, [rids, lanes])
    pltpu.sync_copy(xrows_v, x_out.at[pl.ds(base, _RPW)])
    pltpu.sync_copy(tsel_v, t_out.at[pl.ds(base, _RPW)])


def _run_gather(idx_flat, mem_x, mem_t16):
    mesh = plsc.VectorSubcoreMesh(core_axis_name="c", subcore_axis_name="s")
    f = functools.partial(
        pl.kernel,
        mesh=mesh,
        out_type=[
            jax.ShapeDtypeStruct((_ROWS, C_SIZE), jnp.float32),
            jax.ShapeDtypeStruct((_ROWS,), jnp.float32),
        ],
        scratch_types=[
            pltpu.VMEM((_RPW,), jnp.int32),
            pltpu.VMEM((_RPW,), jnp.int32),
            pltpu.VMEM((_RPW, C_SIZE), jnp.float32),
            pltpu.VMEM((_RPW, _TW), jnp.float32),
            pltpu.VMEM((_RPW,), jnp.float32),
            pltpu.SemaphoreType.DMA,
        ],
        compiler_params=pltpu.CompilerParams(
            use_tc_tiling_on_sc=False, needs_layout_passes=False),
    )(_gather_body)
    return f(mem_x, mem_t16, idx_flat)


def _ln(x, g, b, eps=1e-5):
    mu = jnp.mean(x, axis=-1, keepdims=True)
    xc = x - mu
    var = jnp.mean(xc * xc, axis=-1, keepdims=True)
    return xc / jnp.sqrt(var + eps) * g + b


def _head_body(vals_ref, t_ref, x_ref, wqkv_ref, bqkv_ref, gq_ref, bq_ref,
               gm_ref, bm_ref, w1_ref, b1_ref, w2_ref, b2_ref, p1_ref,
               pb1_ref, p2_ref, pb2_ref, out_ref, qkv_sr, att_sr):
    md = jnp.concatenate([vals_ref[...], t_ref[...], x_ref[...]], axis=1)

    qkv = jnp.dot(md, wqkv_ref[...],
                  preferred_element_type=jnp.float32) + bqkv_ref[...]
    qkv = _ln(qkv, gq_ref[...], bq_ref[...])
    qkv_sr[...] = qkv

    scale = KEY_SIZE ** -0.5

    def attend(b, _):
        blk = qkv_sr[pl.ds(b * TOPK, TOPK), :]
        q = blk[:, :KEY_SIZE] * scale
        kk = blk[:, KEY_SIZE:2 * KEY_SIZE]
        v = blk[:, 2 * KEY_SIZE:]
        w = jnp.dot(q, kk.T, preferred_element_type=jnp.float32)
        w = w - jnp.max(w, axis=1, keepdims=True)
        w = jnp.exp(w)
        w = w / jnp.sum(w, axis=1, keepdims=True)
        att_sr[pl.ds(b * TOPK, TOPK), :] = jnp.dot(
            w, v, preferred_element_type=jnp.float32)
        return 0

    lax.fori_loop(0, B, attend, 0)

    gm = gm_ref[...]
    bm = bm_ref[...]
    mem = _ln(md + att_sr[...], gm, bm)
    h = jnp.dot(jnp.maximum(jnp.dot(mem, w1_ref[...],
                                    preferred_element_type=jnp.float32)
                            + b1_ref[...], 0.0),
                w2_ref[...], preferred_element_type=jnp.float32) + b2_ref[...]
    mem = _ln(h + mem, gm, bm)

    rows = lax.broadcasted_iota(jnp.int32, (B, _ROWS), 1)
    qsel = lax.broadcasted_iota(jnp.int32, (B, _ROWS), 0)
    pool = jnp.where(rows // TOPK == qsel, 1.0 / TOPK, 0.0)
    pooled = jnp.dot(pool, mem, preferred_element_type=jnp.float32)

    c_prime = jnp.dot(
        jnp.maximum(jnp.dot(pooled, p1_ref[...],
                            preferred_element_type=jnp.float32)
                    + pb1_ref[...], 0.0),
        p2_ref[...], preferred_element_type=jnp.float32) + pb2_ref[...]
    out_ref[...] = c_prime


def _run_head(vals, tvals, x_rows, W_qkv, b_qkv, ln_qkv_g, ln_qkv_b,
              ln_mem_g, ln_mem_b, W_mlp1, b_mlp1, W_mlp2, b_mlp2,
              W_proj1, b_proj1, W_proj2, b_proj2):
    args = (vals, tvals, x_rows, W_qkv, b_qkv.reshape(1, -1),
            ln_qkv_g.reshape(1, -1), ln_qkv_b.reshape(1, -1),
            ln_mem_g.reshape(1, -1), ln_mem_b.reshape(1, -1),
            W_mlp1, b_mlp1.reshape(1, -1), W_mlp2, b_mlp2.reshape(1, -1),
            W_proj1, b_proj1.reshape(1, -1), W_proj2, b_proj2.reshape(1, -1))
    return pl.pallas_call(
        _head_body,
        out_shape=jax.ShapeDtypeStruct((B, C_SIZE), jnp.float32),
        scratch_shapes=[
            pltpu.VMEM((_ROWS, QKV_SIZE), jnp.float32),
            pltpu.VMEM((_ROWS, VALUE_SIZE), jnp.float32),
        ],
    )(*args)


def kernel(c, k, mem_c, mem_t, mem_x, W_qkv, b_qkv, ln_qkv_g, ln_qkv_b,
           ln_mem_g, ln_mem_b, W_mlp1, b_mlp1, W_mlp2, b_mlp2,
           W_proj1, b_proj1, W_proj2, b_proj2):
    vals, idx = _run_topk(c, mem_c)
    x_rows, t_flat = _run_gather(idx.reshape(_ROWS), mem_x,
                                 mem_t.reshape(N // _TW, _TW))
    vals = vals.reshape(_ROWS, 1)
    tvals = t_flat.reshape(_ROWS, 1)
    return _run_head(vals, tvals, x_rows, W_qkv, b_qkv, ln_qkv_g, ln_qkv_b,
                     ln_mem_g, ln_mem_b, W_mlp1, b_mlp1, W_mlp2, b_mlp2,
                     W_proj1, b_proj1, W_proj2, b_proj2)
